# baseline (speedup 1.0000x reference)
import functools

import jax
import jax.numpy as jnp
from jax import lax
from jax.experimental import pallas as pl
from jax.experimental.pallas import tpu as pltpu
from jax.experimental.pallas import tpu_sc as plsc

B = 16384
D = 16
ROWS = 1000001
NS = 16
CH = 128
KC = B // (NS * CH)
LANES = 16


def _sc_body(uoff_hbm, moff_hbm, uflat_hbm, mflat_hbm, ubias_hbm, mbias_hbm,
             gb_hbm, out_hbm,
             uoff_v, moff_v, u_v, m_v, ub_v, mb_v, out_v,
             acc_v, partials_v, gb_v, shared, sem, bsem):
    sid = lax.axis_index("s")

    pltpu.sync_copy(uoff_hbm.at[sid], uoff_v)
    pltpu.sync_copy(moff_hbm.at[sid], moff_v)
    pltpu.sync_copy(gb_hbm, gb_v)

    for j in range(KC):
        pltpu.async_copy(ubias_hbm.at[uoff_v.at[0, j]], ub_v.at[j], bsem)
        pltpu.async_copy(mbias_hbm.at[moff_v.at[0, j]], mb_v.at[j], bsem)

    def fire_u(t, carry):
        d = t // KC
        j = t - d * KC
        pltpu.async_copy(uflat_hbm.at[uoff_v.at[d, j]], u_v.at[d, j], sem)
        return carry

    def fire_m(t, carry):
        d = t // KC
        j = t - d * KC
        pltpu.async_copy(mflat_hbm.at[moff_v.at[d, j]], m_v.at[d, j], sem)
        return carry

    lax.fori_loop(0, D * KC, fire_u, 0)
    lax.fori_loop(0, D * KC, fire_m, 0)

    def drain_u(t, carry):
        d = t // KC
        j = t - d * KC
        pltpu.make_async_copy(uflat_hbm.at[uoff_v.at[d, j]], u_v.at[d, j], sem).wait()
        return carry

    def drain_m(t, carry):
        d = t // KC
        j = t - d * KC
        pltpu.make_async_copy(mflat_hbm.at[moff_v.at[d, j]], m_v.at[d, j], sem).wait()
        return carry

    lax.fori_loop(0, D * KC, drain_u, 0)
    lax.fori_loop(0, D * KC, drain_m, 0)

    def dot_step(t, acc):
        d = t // KC
        j = t - d * KC
        for q in range(CH // LANES):
            sl = pl.ds(q * LANES, LANES)
            acc = acc + u_v[d, j, sl] * m_v[d, j, sl]
        return acc

    acc = lax.fori_loop(0, D * KC, dot_step, jnp.zeros((LANES,), jnp.float32))
    acc_v[...] = acc

    pltpu.sync_copy(acc_v, shared.at[sid])
    plsc.subcore_barrier()
    pltpu.sync_copy(shared, partials_v)
    tot = jnp.zeros((LANES,), jnp.float32)
    for t in range(NS):
        tot = tot + partials_v[t, :]
    lane = lax.iota(jnp.int32, LANES)
    for sh in (1, 2, 4, 8):
        acc_v[...] = tot
        tot = tot + plsc.load_gather(acc_v, [lane ^ sh])

    base = tot + gb_v[...]

    for j in range(KC):
        pltpu.make_async_copy(ubias_hbm.at[uoff_v.at[0, j]], ub_v.at[j], bsem).wait()
        pltpu.make_async_copy(mbias_hbm.at[moff_v.at[0, j]], mb_v.at[j], bsem).wait()
    for j in range(KC):
        for q in range(CH // LANES):
            sl = pl.ds(q * LANES, LANES)
            out_v[j, sl] = ub_v[j, sl] + mb_v[j, sl] + base
    pltpu.sync_copy(out_v, out_hbm.at[sid])


@jax.jit
def _run(uoff, moff, uflat, mflat, ubias, mbias, gb16):
    mesh = plsc.VectorSubcoreMesh(core_axis_name="c", subcore_axis_name="s",
                                  num_cores=1)
    f = pl.kernel(
        _sc_body,
        out_type=jax.ShapeDtypeStruct((NS, KC, CH), jnp.float32),
        mesh=mesh,
        scratch_types=[
            pltpu.VMEM((D, KC, CH), jnp.int32),
            pltpu.VMEM((D, KC, CH), jnp.int32),
            pltpu.VMEM((D, KC, CH), jnp.float32),
            pltpu.VMEM((D, KC, CH), jnp.float32),
            pltpu.VMEM((KC, CH), jnp.float32),
            pltpu.VMEM((KC, CH), jnp.float32),
            pltpu.VMEM((KC, CH), jnp.float32),
            pltpu.VMEM((LANES,), jnp.float32),
            pltpu.VMEM((NS, LANES), jnp.float32),
            pltpu.VMEM((LANES,), jnp.float32),
            pltpu.VMEM_SHARED((NS, LANES), jnp.float32),
            pltpu.SemaphoreType.DMA,
            pltpu.SemaphoreType.DMA,
        ],
        compiler_params=pltpu.CompilerParams(needs_layout_passes=False,
                                             use_tc_tiling_on_sc=False),
    )
    return f(uoff, moff, uflat, mflat, ubias, mbias, gb16)


def kernel(inputs, user_embedding, movie_embedding, user_bias_table,
           movie_bias_table, global_bias):
    idx = inputs.astype(jnp.int32)
    dim_off = (jnp.arange(D, dtype=jnp.int32) * ROWS)[None, :, None, None]
    uoff = idx[:, 0].reshape(NS, 1, KC, CH) + dim_off
    moff = idx[:, 1].reshape(NS, 1, KC, CH) + dim_off
    uflat = user_embedding.T.reshape(-1)
    mflat = movie_embedding.T.reshape(-1)
    ubias = user_bias_table.reshape(-1)
    mbias = movie_bias_table.reshape(-1)
    gb16 = jnp.broadcast_to(global_bias.astype(jnp.float32), (LANES,))
    out = _run(uoff, moff, uflat, mflat, ubias, mbias, gb16)
    return out.reshape(B)


# speedup vs baseline: 4.0841x; 4.0841x over previous
import functools

import jax
import jax.numpy as jnp
from jax import lax
from jax.experimental import pallas as pl
from jax.experimental.pallas import tpu as pltpu
from jax.experimental.pallas import tpu_sc as plsc

B = 16384
D = 16
PACK = 128 // D
NS = 16
CH = 128
KC = B // (NS * CH)
LANES = 16


def _sc_body(uidx_hbm, midx_hbm, upack_hbm, mpack_hbm, ubias_hbm, mbias_hbm,
             gb_hbm, out_hbm,
             uidx_v, midx_v, uk_v, mk_v, uo_v, mo_v,
             ubuf_v, mbuf_v, ub_v, mb_v, out_v,
             acc_v, partials_v, gb_v, shared, sem, bsem):
    sid = lax.axis_index("s")

    pltpu.sync_copy(uidx_hbm.at[sid], uidx_v)
    pltpu.sync_copy(midx_hbm.at[sid], midx_v)
    pltpu.sync_copy(gb_hbm, gb_v)

    for j in range(KC):
        for q in range(CH // LANES):
            sl = pl.ds(q * LANES, LANES)
            ru = uidx_v[j, sl]
            rm = midx_v[j, sl]
            uk_v[j, sl] = lax.shift_right_logical(ru, 3)
            mk_v[j, sl] = lax.shift_right_logical(rm, 3)
            uo_v[j, sl] = lax.shift_left(ru & 7, 4)
            mo_v[j, sl] = lax.shift_left(rm & 7, 4)

    for j in range(KC):
        pltpu.async_copy(ubias_hbm.at[uidx_v.at[j]], ub_v.at[j], bsem)
        pltpu.async_copy(mbias_hbm.at[midx_v.at[j]], mb_v.at[j], bsem)

    def fire(j, sel):
        pltpu.async_copy(upack_hbm.at[uk_v.at[j]], ubuf_v.at[sel], sem)
        pltpu.async_copy(mpack_hbm.at[mk_v.at[j]], mbuf_v.at[sel], sem)

    def drain(j, sel):
        pltpu.make_async_copy(upack_hbm.at[uk_v.at[j]], ubuf_v.at[sel], sem).wait()
        pltpu.make_async_copy(mpack_hbm.at[mk_v.at[j]], mbuf_v.at[sel], sem).wait()

    fire(0, 0)
    acc = jnp.zeros((LANES,), jnp.float32)
    for j in range(KC):
        sel = j % 2
        drain(j, sel)
        if j + 1 < KC:
            fire(j + 1, 1 - sel)

        def dot_step(q, a, j=j, sel=sel):
            uo16 = uo_v[j, pl.ds(q * LANES, LANES)]
            mo16 = mo_v[j, pl.ds(q * LANES, LANES)]
            for t in range(LANES):
                i = q * LANES + t
                a = a + (ubuf_v[sel, i, pl.ds(uo16[t], LANES)]
                         * mbuf_v[sel, i, pl.ds(mo16[t], LANES)])
            return a

        acc = lax.fori_loop(0, CH // LANES, dot_step, acc)
    acc_v[...] = acc

    pltpu.sync_copy(acc_v, shared.at[sid])
    plsc.subcore_barrier()
    pltpu.sync_copy(shared, partials_v)
    tot = jnp.zeros((LANES,), jnp.float32)
    for t in range(NS):
        tot = tot + partials_v[t, :]
    lane = lax.iota(jnp.int32, LANES)
    for sh in (1, 2, 4, 8):
        acc_v[...] = tot
        tot = tot + plsc.load_gather(acc_v, [lane ^ sh])

    base = tot + gb_v[...]

    for j in range(KC):
        pltpu.make_async_copy(ubias_hbm.at[uidx_v.at[j]], ub_v.at[j], bsem).wait()
        pltpu.make_async_copy(mbias_hbm.at[midx_v.at[j]], mb_v.at[j], bsem).wait()
    for j in range(KC):
        for q in range(CH // LANES):
            sl = pl.ds(q * LANES, LANES)
            out_v[j, sl] = ub_v[j, sl] + mb_v[j, sl] + base
    pltpu.sync_copy(out_v, out_hbm.at[sid])


@jax.jit
def _run(uidx, midx, upack, mpack, ubias, mbias, gb16):
    mesh = plsc.VectorSubcoreMesh(core_axis_name="c", subcore_axis_name="s",
                                  num_cores=1)
    f = pl.kernel(
        _sc_body,
        out_type=jax.ShapeDtypeStruct((NS, KC, CH), jnp.float32),
        mesh=mesh,
        scratch_types=[
            pltpu.VMEM((KC, CH), jnp.int32),
            pltpu.VMEM((KC, CH), jnp.int32),
            pltpu.VMEM((KC, CH), jnp.int32),
            pltpu.VMEM((KC, CH), jnp.int32),
            pltpu.VMEM((KC, CH), jnp.int32),
            pltpu.VMEM((KC, CH), jnp.int32),
            pltpu.VMEM((2, CH, 128), jnp.float32),
            pltpu.VMEM((2, CH, 128), jnp.float32),
            pltpu.VMEM((KC, CH), jnp.float32),
            pltpu.VMEM((KC, CH), jnp.float32),
            pltpu.VMEM((KC, CH), jnp.float32),
            pltpu.VMEM((LANES,), jnp.float32),
            pltpu.VMEM((NS, LANES), jnp.float32),
            pltpu.VMEM((LANES,), jnp.float32),
            pltpu.VMEM_SHARED((NS, LANES), jnp.float32),
            pltpu.SemaphoreType.DMA,
            pltpu.SemaphoreType.DMA,
        ],
        compiler_params=pltpu.CompilerParams(needs_layout_passes=False,
                                             use_tc_tiling_on_sc=False),
    )
    return f(uidx, midx, upack, mpack, ubias, mbias, gb16)


BKR = 1024
PG = 123
PKR = PG * BKR


def _pack_body(in_ref, out_ref):
    x = in_ref[...]
    y = jnp.swapaxes(x, 0, 1)
    y3 = y.reshape(BKR, PACK, D)
    for s in range(PACK):
        out_ref[:, s * D:(s + 1) * D] = y3[:, s, :]


@jax.jit
def _pack(table_t):
    return pl.pallas_call(
        _pack_body,
        grid=(PG,),
        in_specs=[pl.BlockSpec((D, PACK * BKR), lambda g: (0, g))],
        out_specs=pl.BlockSpec((BKR, PACK * D), lambda g: (g, 0)),
        out_shape=jax.ShapeDtypeStruct((PKR, PACK * D), jnp.float32),
    )(table_t)


def kernel(inputs, user_embedding, movie_embedding, user_bias_table,
           movie_bias_table, global_bias):
    idx = inputs.astype(jnp.int32)
    uidx = idx[:, 0].reshape(NS, KC, CH)
    midx = idx[:, 1].reshape(NS, KC, CH)
    upack = _pack(user_embedding.T)
    mpack = _pack(movie_embedding.T)
    ubias = user_bias_table.reshape(-1)
    mbias = movie_bias_table.reshape(-1)
    gb16 = jnp.broadcast_to(global_bias.astype(jnp.float32), (LANES,))
    out = _run(uidx, midx, upack, mpack, ubias, mbias, gb16)
    return out.reshape(B)


# speedup vs baseline: 9.6984x; 2.3747x over previous
import functools

import jax
import jax.numpy as jnp
from jax import lax
from jax.experimental import pallas as pl
from jax.experimental.pallas import tpu as pltpu
from jax.experimental.pallas import tpu_sc as plsc

B = 16384
D = 16
PACK = 8
NS = 16
BPW = B // NS
CH = 128
KC = BPW // CH
LANES = 16
W2 = 131072
WB = 2048
PG = W2 // WB


def _pack_body(*refs):
    ins, out_ref = refs[:PACK], refs[PACK]
    xx = jnp.concatenate([r[...] for r in ins], axis=0)
    out_ref[...] = jnp.swapaxes(xx, 0, 1)


@jax.jit
def _pack(table_t):
    last_blk = (1000001 - 1) // WB
    in_specs = [
        pl.BlockSpec((D, WB),
                     functools.partial(
                         lambda g, p=0: (0, jnp.minimum(p * PG + g, last_blk)),
                         p=p))
        for p in range(PACK)
    ]
    return pl.pallas_call(
        _pack_body,
        grid=(PG,),
        in_specs=in_specs,
        out_specs=pl.BlockSpec((WB, PACK * D), lambda g: (g, 0)),
        out_shape=jax.ShapeDtypeStruct((W2, PACK * D), jnp.float32),
    )(*([table_t] * PACK))


def _sc_body(uidx_hbm, midx_hbm, uemb_hbm, memb_hbm, ubias_hbm, mbias_hbm,
             gb_hbm, out_hbm,
             uidx_v, midx_v, uk_v, mk_v, uo_v, mo_v,
             ubuf_v, mbuf_v, ub_v, mb_v, out_v,
             acc_v, partials_v, gb_v, shared, sem, bsem):
    sid = lax.axis_index("s")

    pltpu.sync_copy(uidx_hbm.at[sid], uidx_v)
    pltpu.sync_copy(midx_hbm.at[sid], midx_v)
    pltpu.sync_copy(gb_hbm, gb_v)

    for j in range(KC):
        pltpu.async_copy(ubias_hbm.at[uidx_v.at[j]], ub_v.at[j], bsem)
        pltpu.async_copy(mbias_hbm.at[midx_v.at[j]], mb_v.at[j], bsem)

    for j in range(KC):
        for q in range(CH // LANES):
            sl = pl.ds(q * LANES, LANES)
            ru = uidx_v[j, sl]
            rm = midx_v[j, sl]
            uk_v[j, sl] = ru & (W2 - 1)
            mk_v[j, sl] = rm & (W2 - 1)
            uo_v[j, sl] = lax.shift_left(lax.shift_right_logical(ru, 17), 4)
            mo_v[j, sl] = lax.shift_left(lax.shift_right_logical(rm, 17), 4)

    def fire_rows(j, sel):
        pltpu.async_copy(uemb_hbm.at[uk_v.at[j]], ubuf_v.at[sel], sem)
        pltpu.async_copy(memb_hbm.at[mk_v.at[j]], mbuf_v.at[sel], sem)

    def drain_rows(j, sel):
        pltpu.make_async_copy(uemb_hbm.at[uk_v.at[j]], ubuf_v.at[sel], sem).wait()
        pltpu.make_async_copy(memb_hbm.at[mk_v.at[j]], mbuf_v.at[sel], sem).wait()

    fire_rows(0, 0)
    acc = jnp.zeros((LANES,), jnp.float32)
    for j in range(KC):
        sel = j % 2
        drain_rows(j, sel)
        if j + 1 < KC:
            fire_rows(j + 1, 1 - sel)

        def dot_step(q, a, j=j, sel=sel):
            uo16 = uo_v[j, pl.ds(q * LANES, LANES)]
            mo16 = mo_v[j, pl.ds(q * LANES, LANES)]
            for t in range(LANES):
                i = q * LANES + t
                a = a + (ubuf_v[sel, i, pl.ds(uo16[t], LANES)]
                         * mbuf_v[sel, i, pl.ds(mo16[t], LANES)])
            return a

        acc = lax.fori_loop(0, CH // LANES, dot_step, acc)
    acc_v[...] = acc

    pltpu.sync_copy(acc_v, shared.at[sid])
    plsc.subcore_barrier()
    pltpu.sync_copy(shared, partials_v)
    tot = jnp.zeros((LANES,), jnp.float32)
    for t in range(NS):
        tot = tot + partials_v[t, :]
    lane = lax.iota(jnp.int32, LANES)
    for sh in (1, 2, 4, 8):
        acc_v[...] = tot
        tot = tot + plsc.load_gather(acc_v, [lane ^ sh])

    base = tot + gb_v[...]

    for j in range(KC):
        pltpu.make_async_copy(ubias_hbm.at[uidx_v.at[j]], ub_v.at[j], bsem).wait()
        pltpu.make_async_copy(mbias_hbm.at[midx_v.at[j]], mb_v.at[j], bsem).wait()
    for j in range(KC):
        for q in range(CH // LANES):
            sl = pl.ds(q * LANES, LANES)
            out_v[j, sl] = ub_v[j, sl] + mb_v[j, sl] + base
    pltpu.sync_copy(out_v, out_hbm.at[sid])


@jax.jit
def _run(uidx, midx, uembt, membt, ubias, mbias, gb16):
    mesh = plsc.VectorSubcoreMesh(core_axis_name="c", subcore_axis_name="s",
                                  num_cores=1)
    f = pl.kernel(
        _sc_body,
        out_type=jax.ShapeDtypeStruct((NS, KC, CH), jnp.float32),
        mesh=mesh,
        scratch_types=[
            pltpu.VMEM((KC, CH), jnp.int32),
            pltpu.VMEM((KC, CH), jnp.int32),
            pltpu.VMEM((KC, CH), jnp.int32),
            pltpu.VMEM((KC, CH), jnp.int32),
            pltpu.VMEM((KC, CH), jnp.int32),
            pltpu.VMEM((KC, CH), jnp.int32),
            pltpu.VMEM((2, CH, PACK * D), jnp.float32),
            pltpu.VMEM((2, CH, PACK * D), jnp.float32),
            pltpu.VMEM((KC, CH), jnp.float32),
            pltpu.VMEM((KC, CH), jnp.float32),
            pltpu.VMEM((KC, CH), jnp.float32),
            pltpu.VMEM((LANES,), jnp.float32),
            pltpu.VMEM((NS, LANES), jnp.float32),
            pltpu.VMEM((LANES,), jnp.float32),
            pltpu.VMEM_SHARED((NS, LANES), jnp.float32),
            pltpu.SemaphoreType.DMA,
            pltpu.SemaphoreType.DMA,
        ],
        compiler_params=pltpu.CompilerParams(needs_layout_passes=False,
                                             use_tc_tiling_on_sc=False),
    )
    return f(uidx, midx, uembt, membt, ubias, mbias, gb16)


def kernel(inputs, user_embedding, movie_embedding, user_bias_table,
           movie_bias_table, global_bias):
    idx = inputs.astype(jnp.int32)
    uidx = idx[:, 0].reshape(NS, KC, CH)
    midx = idx[:, 1].reshape(NS, KC, CH)
    ubias = user_bias_table.reshape(-1)
    mbias = movie_bias_table.reshape(-1)
    gb16 = jnp.broadcast_to(global_bias.astype(jnp.float32), (LANES,))
    upack = _pack(user_embedding.T)
    mpack = _pack(movie_embedding.T)
    out = _run(uidx, midx, upack, mpack, ubias, mbias, gb16)
    return out.reshape(B)
